# Initial kernel scaffold; baseline (speedup 1.0000x reference)
import jax
import jax.numpy as jnp
from jax.experimental import pallas as pl


def kernel(w1, b1, wl, bl, head_w1, head_b1, head_w2, head_b2, head_w3, head_b3, sc_x, sc_adj, fc_x, fc_adj, pool_mat):
    raise NotImplementedError("write your pallas kernel here")


# speedup vs baseline: 1.3175x; 1.3175x over previous
import jax
import jax.numpy as jnp
from jax.experimental import pallas as pl
from jax.experimental.pallas import tpu as pltpu

_NUM_CLASSES_OUT = 64


def _fused_body(w1_ref, b1_ref, wl_ref, bl_ref,
                hw1_ref, hb1_ref, hw2_ref, hb2_ref, hw3_ref, hb3_ref,
                scx_ref, scadj_hbm, fcx_ref, fcadj_hbm, pool_ref,
                o_ref, a16, stage, sem, h, pooled):
    n = a16.shape[0]
    tile = stage.shape[1]
    nt = n // tile
    f_pad = w1_ref.shape[1] // 2
    h_pad = w1_ref.shape[2]
    num_extra = wl_ref.shape[1]

    def run_branch(adj_hbm, x_ref, bidx):
        w1b = w1_ref[bidx]
        b1b = b1_ref[bidx]

        def dma_start(t):
            pltpu.make_async_copy(adj_hbm.at[pl.ds(t * tile, tile)],
                                  stage.at[t % 2], sem.at[t % 2]).start()

        def dma_wait(t):
            pltpu.make_async_copy(adj_hbm.at[pl.ds(t * tile, tile)],
                                  stage.at[t % 2], sem.at[t % 2]).wait()

        dma_start(0)
        for t in range(nt):
            if t + 1 < nt:
                dma_start(t + 1)
            dma_wait(t)
            a_t = stage[t % 2].astype(jnp.bfloat16)
            a16[pl.ds(t * tile, tile), :] = a_t
            agg = jnp.dot(a_t, x_ref[...],
                          preferred_element_type=jnp.float32)
            z = (jnp.dot(agg.astype(jnp.bfloat16), w1b[:f_pad],
                         preferred_element_type=jnp.float32)
                 + jnp.dot(x_ref[pl.ds(t * tile, tile), :], w1b[f_pad:],
                           preferred_element_type=jnp.float32)
                 + b1b)
            h[pl.ds(t * tile, tile), :] = jnp.maximum(z, 0.0).astype(jnp.bfloat16)

        for layer in range(num_extra):
            wlb = wl_ref[bidx, layer]
            blb = bl_ref[bidx, layer]
            agg = jnp.dot(a16[...], h[...],
                          preferred_element_type=jnp.float32)
            z = (jnp.dot(agg.astype(jnp.bfloat16), wlb[:h_pad],
                         preferred_element_type=jnp.float32)
                 + jnp.dot(h[...], wlb[h_pad:],
                           preferred_element_type=jnp.float32)
                 + blb)
            h[...] = jnp.maximum(z, 0.0).astype(jnp.bfloat16)

        pooled[:, bidx * h_pad:(bidx + 1) * h_pad] = jnp.dot(
            pool_ref[...], h[...], preferred_element_type=jnp.float32)

    run_branch(scadj_hbm, scx_ref, 0)
    run_branch(fcadj_hbm, fcx_ref, 1)

    t1 = jnp.maximum(jnp.dot(pooled[...], hw1_ref[...],
                             preferred_element_type=jnp.float32)
                     + hb1_ref[...], 0.0)
    t2 = jnp.maximum(jnp.dot(t1, hw2_ref[...],
                             preferred_element_type=jnp.float32)
                     + hb2_ref[...], 0.0)
    logits = jnp.dot(t2, hw3_ref[...],
                     preferred_element_type=jnp.float32) + hb3_ref[...]
    col = jax.lax.broadcasted_iota(jnp.int32, logits.shape, 1)
    logits = jnp.where(col < _NUM_CLASSES_OUT, logits, -1e30)
    m = jnp.max(logits, axis=-1, keepdims=True)
    z = logits - m
    lse = jnp.log(jnp.sum(jnp.exp(z), axis=-1, keepdims=True))
    o_ref[...] = (z - lse)[:, :_NUM_CLASSES_OUT]


def kernel(w1, b1, wl, bl, head_w1, head_b1, head_w2, head_b2,
           head_w3, head_b3, sc_x, sc_adj, fc_x, fc_adj, pool_mat):
    n = sc_x.shape[0]
    g = pool_mat.shape[0]
    h_pad = w1.shape[2]

    tile = next(c for c in (512, 256, 128, 64, 32, 16, 8, 1) if n % c == 0)
    tile = min(tile, n)

    scx16 = sc_x.astype(jnp.bfloat16)
    fcx16 = fc_x.astype(jnp.bfloat16)
    pool16 = pool_mat.astype(jnp.bfloat16)

    vmem = pl.BlockSpec(memory_space=pltpu.MemorySpace.VMEM)
    hbm = pl.BlockSpec(memory_space=pl.ANY)

    out = pl.pallas_call(
        _fused_body,
        out_shape=jax.ShapeDtypeStruct((g, _NUM_CLASSES_OUT), jnp.float32),
        in_specs=[vmem, vmem, vmem, vmem,
                  vmem, vmem, vmem, vmem, vmem, vmem,
                  vmem, hbm, vmem, hbm, vmem],
        out_specs=vmem,
        scratch_shapes=[
            pltpu.VMEM((n, n), jnp.bfloat16),
            pltpu.VMEM((2, tile, n), jnp.float32),
            pltpu.SemaphoreType.DMA((2,)),
            pltpu.VMEM((n, h_pad), jnp.bfloat16),
            pltpu.VMEM((g, 2 * h_pad), jnp.float32),
        ],
        name="graphsage_fused",
    )(w1, b1, wl, bl, head_w1, head_b1, head_w2, head_b2, head_w3, head_b3,
      scx16, sc_adj, fcx16, fc_adj, pool16)
    return out


# speedup vs baseline: 1.5057x; 1.1428x over previous
import jax
import jax.numpy as jnp
from jax.experimental import pallas as pl
from jax.experimental.pallas import tpu as pltpu

_NUM_CLASSES_OUT = 64


def _fused_body(w1_ref, b1_ref, wl_ref, bl_ref,
                hw1_ref, hb1_ref, hw2_ref, hb2_ref, hw3_ref, hb3_ref,
                scx_ref, scadj_hbm, fcx_ref, fcadj_hbm, pool_ref,
                o_ref, abuf_a, abuf_b, sem, h, pooled):
    n = abuf_a.shape[0]
    tile = abuf_a.shape[0] // sem.shape[1]
    nt = sem.shape[1]
    f_pad = w1_ref.shape[1] // 2
    h_pad = w1_ref.shape[2]
    num_extra = wl_ref.shape[1]

    def copy(adj_hbm, abuf, bidx, t):
        return pltpu.make_async_copy(
            adj_hbm.at[pl.ds(t * tile, tile)],
            abuf.at[pl.ds(t * tile, tile)],
            sem.at[bidx, t])

    for t in range(nt):
        copy(scadj_hbm, abuf_a, 0, t).start()
    for t in range(nt):
        copy(fcadj_hbm, abuf_b, 1, t).start()

    def run_branch(adj_hbm, abuf, x_ref, bidx):
        w1f = w1_ref[bidx].astype(jnp.float32)
        b1b = b1_ref[bidx]

        for t in range(nt):
            copy(adj_hbm, abuf, bidx, t).wait()
            a_t = abuf[pl.ds(t * tile, tile), :]
            agg = jnp.dot(a_t, x_ref[...],
                          preferred_element_type=jnp.float32)
            z = (jnp.dot(agg, w1f[:f_pad],
                         preferred_element_type=jnp.float32)
                 + jnp.dot(x_ref[pl.ds(t * tile, tile), :], w1f[f_pad:],
                           preferred_element_type=jnp.float32)
                 + b1b)
            h[pl.ds(t * tile, tile), :] = jnp.maximum(z, 0.0)

        for layer in range(num_extra):
            wlf = wl_ref[bidx, layer].astype(jnp.float32)
            blb = bl_ref[bidx, layer]
            agg = jnp.dot(abuf[...], h[...],
                          preferred_element_type=jnp.float32)
            z = (jnp.dot(agg, wlf[:h_pad],
                         preferred_element_type=jnp.float32)
                 + jnp.dot(h[...], wlf[h_pad:],
                           preferred_element_type=jnp.float32)
                 + blb)
            h[...] = jnp.maximum(z, 0.0)

        pooled[:, bidx * h_pad:(bidx + 1) * h_pad] = jnp.dot(
            pool_ref[...], h[...], preferred_element_type=jnp.float32)

    run_branch(scadj_hbm, abuf_a, scx_ref, 0)
    run_branch(fcadj_hbm, abuf_b, fcx_ref, 1)

    t1 = jnp.maximum(jnp.dot(pooled[...], hw1_ref[...],
                             preferred_element_type=jnp.float32)
                     + hb1_ref[...], 0.0)
    t2 = jnp.maximum(jnp.dot(t1, hw2_ref[...],
                             preferred_element_type=jnp.float32)
                     + hb2_ref[...], 0.0)
    logits = jnp.dot(t2, hw3_ref[...],
                     preferred_element_type=jnp.float32) + hb3_ref[...]
    col = jax.lax.broadcasted_iota(jnp.int32, logits.shape, 1)
    logits = jnp.where(col < _NUM_CLASSES_OUT, logits, -1e30)
    m = jnp.max(logits, axis=-1, keepdims=True)
    z = logits - m
    lse = jnp.log(jnp.sum(jnp.exp(z), axis=-1, keepdims=True))
    o_ref[...] = (z - lse)[:, :_NUM_CLASSES_OUT]


def kernel(w1, b1, wl, bl, head_w1, head_b1, head_w2, head_b2,
           head_w3, head_b3, sc_x, sc_adj, fc_x, fc_adj, pool_mat):
    n = sc_x.shape[0]
    g = pool_mat.shape[0]
    h_pad = w1.shape[2]

    tile = next(c for c in (256, 128, 64, 32, 16, 8, 1) if n % c == 0)

    vmem = pl.BlockSpec(memory_space=pltpu.MemorySpace.VMEM)
    hbm = pl.BlockSpec(memory_space=pl.ANY)

    out = pl.pallas_call(
        _fused_body,
        out_shape=jax.ShapeDtypeStruct((g, _NUM_CLASSES_OUT), jnp.float32),
        in_specs=[vmem, vmem, vmem, vmem,
                  vmem, vmem, vmem, vmem, vmem, vmem,
                  vmem, hbm, vmem, hbm, vmem],
        out_specs=vmem,
        scratch_shapes=[
            pltpu.VMEM((n, n), jnp.float32),
            pltpu.VMEM((n, n), jnp.float32),
            pltpu.SemaphoreType.DMA((2, n // tile)),
            pltpu.VMEM((n, h_pad), jnp.float32),
            pltpu.VMEM((g, 2 * h_pad), jnp.float32),
        ],
        name="graphsage_fused",
    )(w1, b1, wl, bl, head_w1, head_b1, head_w2, head_b2, head_w3, head_b3,
      sc_x, sc_adj, fc_x, fc_adj, pool_mat)
    return out


# speedup vs baseline: 1.5180x; 1.0081x over previous
import jax
import jax.numpy as jnp
from jax.experimental import pallas as pl
from jax.experimental.pallas import tpu as pltpu

_NUM_CLASSES_OUT = 64


def _fused_body(w1_ref, b1_ref, wl_ref, bl_ref,
                hw1_ref, hb1_ref, hw2_ref, hb2_ref, hw3_ref, hb3_ref,
                scx_ref, scadj_hbm, fcx_ref, fcadj_hbm, pool_ref,
                o_ref, abuf_a, abuf_b, sem, h_a, h_b, pooled):
    n = abuf_a.shape[0]
    nt = sem.shape[1]
    tile = n // nt
    f_pad = w1_ref.shape[1] // 2
    h_pad = w1_ref.shape[2]
    num_extra = wl_ref.shape[1]

    def copy(adj_hbm, abuf, bidx, t):
        return pltpu.make_async_copy(
            adj_hbm.at[pl.ds(t * tile, tile)],
            abuf.at[pl.ds(t * tile, tile)],
            sem.at[bidx, t])

    for t in range(nt):
        copy(scadj_hbm, abuf_a, 0, t).start()
    for t in range(nt):
        copy(fcadj_hbm, abuf_b, 1, t).start()

    def layer1(adj_hbm, abuf, x_ref, h, bidx):
        w1f = w1_ref[bidx].astype(jnp.float32)
        b1b = b1_ref[bidx]
        for t in range(nt):
            copy(adj_hbm, abuf, bidx, t).wait()
            a_t = abuf[pl.ds(t * tile, tile), :]
            agg = jnp.dot(a_t, x_ref[...],
                          preferred_element_type=jnp.float32)
            z = (jnp.dot(agg, w1f[:f_pad],
                         preferred_element_type=jnp.float32)
                 + jnp.dot(x_ref[pl.ds(t * tile, tile), :], w1f[f_pad:],
                           preferred_element_type=jnp.float32)
                 + b1b)
            h[pl.ds(t * tile, tile), :] = jnp.maximum(z, 0.0)

    layer1(scadj_hbm, abuf_a, scx_ref, h_a, 0)
    layer1(fcadj_hbm, abuf_b, fcx_ref, h_b, 1)

    for layer in range(num_extra):
        def sage(abuf, h, bidx):
            wlf = wl_ref[bidx, layer].astype(jnp.float32)
            blb = bl_ref[bidx, layer]
            agg = jnp.dot(abuf[...], h[...],
                          preferred_element_type=jnp.float32)
            z = (jnp.dot(agg, wlf[:h_pad],
                         preferred_element_type=jnp.float32)
                 + jnp.dot(h[...], wlf[h_pad:],
                           preferred_element_type=jnp.float32)
                 + blb)
            h[...] = jnp.maximum(z, 0.0)
        sage(abuf_a, h_a, 0)
        sage(abuf_b, h_b, 1)

    pooled[:, 0:h_pad] = jnp.dot(pool_ref[...], h_a[...],
                                 preferred_element_type=jnp.float32)
    pooled[:, h_pad:2 * h_pad] = jnp.dot(pool_ref[...], h_b[...],
                                         preferred_element_type=jnp.float32)

    t1 = jnp.maximum(jnp.dot(pooled[...], hw1_ref[...],
                             preferred_element_type=jnp.float32)
                     + hb1_ref[...], 0.0)
    t2 = jnp.maximum(jnp.dot(t1, hw2_ref[...],
                             preferred_element_type=jnp.float32)
                     + hb2_ref[...], 0.0)
    logits = jnp.dot(t2, hw3_ref[...],
                     preferred_element_type=jnp.float32) + hb3_ref[...]
    col = jax.lax.broadcasted_iota(jnp.int32, logits.shape, 1)
    logits = jnp.where(col < _NUM_CLASSES_OUT, logits, -1e30)
    m = jnp.max(logits, axis=-1, keepdims=True)
    z = logits - m
    lse = jnp.log(jnp.sum(jnp.exp(z), axis=-1, keepdims=True))
    o_ref[...] = (z - lse)[:, :_NUM_CLASSES_OUT]


def kernel(w1, b1, wl, bl, head_w1, head_b1, head_w2, head_b2,
           head_w3, head_b3, sc_x, sc_adj, fc_x, fc_adj, pool_mat):
    n = sc_x.shape[0]
    g = pool_mat.shape[0]
    h_pad = w1.shape[2]

    tile = next(c for c in (256, 128, 64, 32, 16, 8, 1) if n % c == 0)

    vmem = pl.BlockSpec(memory_space=pltpu.MemorySpace.VMEM)
    hbm = pl.BlockSpec(memory_space=pl.ANY)

    out = pl.pallas_call(
        _fused_body,
        out_shape=jax.ShapeDtypeStruct((g, _NUM_CLASSES_OUT), jnp.float32),
        in_specs=[vmem, vmem, vmem, vmem,
                  vmem, vmem, vmem, vmem, vmem, vmem,
                  vmem, hbm, vmem, hbm, vmem],
        out_specs=vmem,
        scratch_shapes=[
            pltpu.VMEM((n, n), jnp.float32),
            pltpu.VMEM((n, n), jnp.float32),
            pltpu.SemaphoreType.DMA((2, n // tile)),
            pltpu.VMEM((n, h_pad), jnp.float32),
            pltpu.VMEM((n, h_pad), jnp.float32),
            pltpu.VMEM((g, 2 * h_pad), jnp.float32),
        ],
        name="graphsage_fused",
    )(w1, b1, wl, bl, head_w1, head_b1, head_w2, head_b2, head_w3, head_b3,
      sc_x, sc_adj, fc_x, fc_adj, pool_mat)
    return out


# speedup vs baseline: 1.6307x; 1.0743x over previous
import jax
import jax.numpy as jnp
from jax.experimental import pallas as pl
from jax.experimental.pallas import tpu as pltpu

_NUM_CLASSES_OUT = 64


def _fused_body(w1_ref, b1_ref, wl_ref, bl_ref,
                hw1_ref, hb1_ref, hw2_ref, hb2_ref, hw3_ref, hb3_ref,
                scx_ref, scadj_hbm, fcx_ref, fcadj_hbm, pool_ref,
                o_ref, abuf_a, abuf_b, a16, sem, h16, pooled):
    n = abuf_a.shape[0]
    nt = sem.shape[1]
    tile = n // nt
    f_pad = w1_ref.shape[1] // 2
    h_pad = w1_ref.shape[2]
    num_extra = wl_ref.shape[1]

    def copy(adj_hbm, abuf, bidx, t):
        return pltpu.make_async_copy(
            adj_hbm.at[pl.ds(t * tile, tile)],
            abuf.at[pl.ds(t * tile, tile)],
            sem.at[bidx, t])

    for t in range(nt):
        copy(scadj_hbm, abuf_a, 0, t).start()
    for t in range(nt):
        copy(fcadj_hbm, abuf_b, 1, t).start()

    pool16 = pool_ref[...].astype(jnp.bfloat16)

    def run_branch(adj_hbm, abuf, x_ref, bidx):
        x16 = x_ref[...].astype(jnp.bfloat16)
        w1b = w1_ref[bidx]
        b1b = b1_ref[bidx]

        for t in range(nt):
            copy(adj_hbm, abuf, bidx, t).wait()
            a_t = abuf[pl.ds(t * tile, tile), :].astype(jnp.bfloat16)
            a16[pl.ds(t * tile, tile), :] = a_t
            agg = jnp.dot(a_t, x16, preferred_element_type=jnp.float32)
            z = (jnp.dot(agg.astype(jnp.bfloat16), w1b[:f_pad],
                         preferred_element_type=jnp.float32)
                 + jnp.dot(x16[t * tile:(t + 1) * tile], w1b[f_pad:],
                           preferred_element_type=jnp.float32)
                 + b1b)
            h16[pl.ds(t * tile, tile), :] = jnp.maximum(z, 0.0).astype(jnp.bfloat16)

        for layer in range(num_extra):
            wlb = wl_ref[bidx, layer]
            blb = bl_ref[bidx, layer]
            agg = jnp.dot(a16[...], h16[...],
                          preferred_element_type=jnp.float32)
            z = (jnp.dot(agg.astype(jnp.bfloat16), wlb[:h_pad],
                         preferred_element_type=jnp.float32)
                 + jnp.dot(h16[...], wlb[h_pad:],
                           preferred_element_type=jnp.float32)
                 + blb)
            h16[...] = jnp.maximum(z, 0.0).astype(jnp.bfloat16)

        pooled[:, bidx * h_pad:(bidx + 1) * h_pad] = jnp.dot(
            pool16, h16[...], preferred_element_type=jnp.float32)

    run_branch(scadj_hbm, abuf_a, scx_ref, 0)
    run_branch(fcadj_hbm, abuf_b, fcx_ref, 1)

    t1 = jnp.maximum(jnp.dot(pooled[...], hw1_ref[...],
                             preferred_element_type=jnp.float32)
                     + hb1_ref[...], 0.0)
    t2 = jnp.maximum(jnp.dot(t1, hw2_ref[...],
                             preferred_element_type=jnp.float32)
                     + hb2_ref[...], 0.0)
    logits = jnp.dot(t2, hw3_ref[...],
                     preferred_element_type=jnp.float32) + hb3_ref[...]
    col = jax.lax.broadcasted_iota(jnp.int32, logits.shape, 1)
    logits = jnp.where(col < _NUM_CLASSES_OUT, logits, -1e30)
    m = jnp.max(logits, axis=-1, keepdims=True)
    z = logits - m
    lse = jnp.log(jnp.sum(jnp.exp(z), axis=-1, keepdims=True))
    o_ref[...] = (z - lse)[:, :_NUM_CLASSES_OUT]


def kernel(w1, b1, wl, bl, head_w1, head_b1, head_w2, head_b2,
           head_w3, head_b3, sc_x, sc_adj, fc_x, fc_adj, pool_mat):
    n = sc_x.shape[0]
    g = pool_mat.shape[0]
    h_pad = w1.shape[2]

    tile = next(c for c in (256, 128, 64, 32, 16, 8, 1) if n % c == 0)

    vmem = pl.BlockSpec(memory_space=pltpu.MemorySpace.VMEM)
    hbm = pl.BlockSpec(memory_space=pl.ANY)

    out = pl.pallas_call(
        _fused_body,
        out_shape=jax.ShapeDtypeStruct((g, _NUM_CLASSES_OUT), jnp.float32),
        in_specs=[vmem, vmem, vmem, vmem,
                  vmem, vmem, vmem, vmem, vmem, vmem,
                  vmem, hbm, vmem, hbm, vmem],
        out_specs=vmem,
        scratch_shapes=[
            pltpu.VMEM((n, n), jnp.float32),
            pltpu.VMEM((n, n), jnp.float32),
            pltpu.VMEM((n, n), jnp.bfloat16),
            pltpu.SemaphoreType.DMA((2, n // tile)),
            pltpu.VMEM((n, h_pad), jnp.bfloat16),
            pltpu.VMEM((g, 2 * h_pad), jnp.float32),
        ],
        name="graphsage_fused",
    )(w1, b1, wl, bl, head_w1, head_b1, head_w2, head_b2, head_w3, head_b3,
      sc_x, sc_adj, fc_x, fc_adj, pool_mat)
    return out
